# Initial kernel scaffold; baseline (speedup 1.0000x reference)
import functools

import jax
import jax.numpy as jnp
from jax import lax
from jax.experimental import pallas as pl
from jax.experimental.pallas import tpu as pltpu
from jax.experimental.pallas import tpu_sc as plsc

N = 10000
NP = 10240
E = 320000
D = 128
NC = 2
NS = 16
NW = NC * NS
EP = 330240
EPW = EP // NW
K = 80
NCHUNK = EPW // K
RPT = NP // NS

_mesh = plsc.VectorSubcoreMesh(core_axis_name="c", subcore_axis_name="s")


def _rsqrt_newton(d):
    dd = jnp.maximum(d, jnp.float32(1e-30))
    i = lax.bitcast_convert_type(dd, jnp.int32)
    i = jnp.int32(0x5F3759DF) - lax.shift_right_arithmetic(i, jnp.int32(1))
    y = lax.bitcast_convert_type(i, jnp.float32)
    half = jnp.float32(0.5) * dd
    for _ in range(3):
        y = y * (jnp.float32(1.5) - half * y * y)
    return jnp.where(d > 0, y, jnp.float32(0.0))


@functools.partial(
    pl.kernel,
    out_type=jax.ShapeDtypeStruct((NC, NP), jnp.float32),
    mesh=_mesh,
    scratch_types=[
        pltpu.VMEM((EPW,), jnp.int32),
        pltpu.VMEM((EPW,), jnp.float32),
        pltpu.VMEM((NP,), jnp.float32),
        pltpu.VMEM((NS, RPT), jnp.float32),
        pltpu.VMEM((RPT,), jnp.float32),
        pltpu.VMEM_SHARED((NS, NP), jnp.float32),
    ],
)
def _deg(col_hbm, ew_hbm, degp_hbm, colv, ewv, degv, tmpv, accv, shared):
    cid = lax.axis_index("c")
    sid = lax.axis_index("s")
    wid = sid * NC + cid
    zeros16 = jnp.zeros((16,), jnp.float32)

    @pl.loop(0, NP, step=16)
    def _(i):
        degv[pl.ds(i, 16)] = zeros16

    pltpu.sync_copy(col_hbm.at[wid], colv)
    pltpu.sync_copy(ew_hbm.at[wid], ewv)

    @pl.loop(0, EPW, step=16)
    def _(e):
        sl = pl.ds(e, 16)
        plsc.addupdate_scatter(degv, [colv[sl]], ewv[sl])

    pltpu.sync_copy(degv, shared.at[sid])
    plsc.subcore_barrier()
    base = sid * RPT
    pltpu.sync_copy(shared.at[:, pl.ds(base, RPT)], tmpv)

    @pl.loop(0, RPT, step=16)
    def _(i):
        sl = pl.ds(i, 16)
        acc = tmpv[0, sl]
        for t in range(1, NS):
            acc = acc + tmpv[t, sl]
        accv[sl] = acc

    pltpu.sync_copy(accv, degp_hbm.at[cid, pl.ds(base, RPT)])


@functools.partial(
    pl.kernel,
    out_type=jax.ShapeDtypeStruct((NC, NP, D), jnp.float32),
    mesh=_mesh,
    scratch_types=[
        pltpu.VMEM((NCHUNK, K), jnp.int32),
        pltpu.VMEM((NCHUNK, K), jnp.int32),
        pltpu.VMEM((NCHUNK, K), jnp.float32),
        pltpu.VMEM((NP,), jnp.float32),
        pltpu.VMEM((NC, NP), jnp.float32),
        pltpu.VMEM((K, D), jnp.float32),
        pltpu.VMEM_SHARED((NP, D), jnp.float32),
    ],
)
def _agg(row_hbm, col_hbm, ew_hbm, degp_hbm, table_hbm, out_hbm,
         rowv, colv, wv, disv, dgv, msgv, acc):
    cid = lax.axis_index("c")
    sid = lax.axis_index("s")
    wid = sid * NC + cid
    zeros16 = jnp.zeros((16,), jnp.float32)

    @pl.loop(0, K)
    def _(r):
        for q in range(D // 16):
            msgv[r, pl.ds(q * 16, 16)] = zeros16

    for j in range(RPT // K):
        pltpu.sync_copy(msgv, acc.at[pl.ds(sid * RPT + j * K, K)])

    pltpu.sync_copy(degp_hbm, dgv)

    @pl.loop(0, NP, step=16)
    def _(i):
        sl = pl.ds(i, 16)
        disv[sl] = _rsqrt_newton(dgv[0, sl] + dgv[1, sl])

    pltpu.sync_copy(row_hbm.at[wid], rowv)
    pltpu.sync_copy(col_hbm.at[wid], colv)
    pltpu.sync_copy(ew_hbm.at[wid], wv)

    @pl.loop(0, NCHUNK)
    def _(c):
        @pl.loop(0, K, step=16)
        def _(j):
            sl = pl.ds(j, 16)
            dr = plsc.load_gather(disv, [rowv[c, sl]])
            dc = plsc.load_gather(disv, [colv[c, sl]])
            wv[c, sl] = wv[c, sl] * dr * dc

    plsc.subcore_barrier()

    @pl.loop(0, NCHUNK)
    def _(c):
        pltpu.sync_copy(table_hbm.at[rowv.at[c]], msgv)
        cvec = jnp.full((16,), c, dtype=jnp.int32)

        @pl.loop(0, K)
        def _(j):
            jvec = jnp.full((16,), j, dtype=jnp.int32)
            wj = plsc.load_gather(wv, [cvec, jvec])
            for q in range(D // 16):
                sl = pl.ds(q * 16, 16)
                msgv[j, sl] = msgv[j, sl] * wj

        pltpu.sync_copy(msgv, acc.at[colv.at[c]], add=True)

    plsc.subcore_barrier()
    pltpu.sync_copy(acc.at[pl.ds(sid * RPT, RPT)],
                    out_hbm.at[cid, pl.ds(sid * RPT, RPT)])


def _mm_body(x_ref, w_ref, o_ref):
    o_ref[...] = jnp.dot(x_ref[...], w_ref[...],
                         preferred_element_type=jnp.float32)


_mm = pl.pallas_call(
    _mm_body, out_shape=jax.ShapeDtypeStruct((NP, D), jnp.float32))


def _relu_mm_body(a_ref, b_ref, w_ref, o_ref):
    h = jnp.maximum(a_ref[0] + a_ref[1] + b_ref[...], 0.0)
    o_ref[...] = jnp.dot(h, w_ref[...], preferred_element_type=jnp.float32)


_relu_mm = pl.pallas_call(
    _relu_mm_body, out_shape=jax.ShapeDtypeStruct((NP, D), jnp.float32))


def _relu_body(a_ref, b_ref, o_ref):
    o_ref[...] = jnp.maximum(a_ref[0] + a_ref[1] + b_ref[...], 0.0)


_relu = pl.pallas_call(
    _relu_body, out_shape=jax.ShapeDtypeStruct((NP, D), jnp.float32))


def kernel(x, edge_index, edge_weight, W1, b1, W2, b2):
    row = edge_index[0].astype(jnp.int32)
    col = edge_index[1].astype(jnp.int32)
    loop = jnp.arange(N, dtype=jnp.int32)
    padi = jnp.full((EP - E - N,), NP - 1, dtype=jnp.int32)
    row_f = jnp.concatenate([row, loop, padi])
    col_f = jnp.concatenate([col, loop, padi])
    ew_f = jnp.concatenate([
        edge_weight.astype(jnp.float32),
        jnp.ones((N,), jnp.float32),
        jnp.zeros((EP - E - N,), jnp.float32),
    ])
    row_c = row_f.reshape(NW, NCHUNK, K)
    col_c = col_f.reshape(NW, NCHUNK, K)
    ew_c = ew_f.reshape(NW, NCHUNK, K)
    x_pad = jnp.concatenate([x, jnp.zeros((NP - N, D), x.dtype)])

    degp = _deg(col_f.reshape(NW, EPW), ew_f.reshape(NW, EPW))
    xw1 = _mm(x_pad, W1)
    agg1 = _agg(row_c, col_c, ew_c, degp, xw1)
    xw2 = _relu_mm(agg1, b1.reshape(1, D), W2)
    agg2 = _agg(row_c, col_c, ew_c, degp, xw2)
    out = _relu(agg2, b2.reshape(1, D))
    return out[:N]


# speedup vs baseline: 9.6670x; 9.6670x over previous
import dataclasses
import functools

import jax
import jax.numpy as jnp
from jax import lax
from jax.experimental import pallas as pl
from jax.experimental.pallas import tpu as pltpu
from jax.experimental.pallas import tpu_sc as plsc

N = 10000
NP = 10240
E = 320000
D = 128
DH = 64
NC = 2
NS = 16
NW = NC * NS
EP = 330240
EPW = EP // NW
EPS = EP // NS
K = 80
NCHUNK = EPS // K
RPT = NP // NS

_mesh = plsc.VectorSubcoreMesh(core_axis_name="c", subcore_axis_name="s")

_sc_params = pltpu.CompilerParams(needs_layout_passes=False,
                                  use_tc_tiling_on_sc=False)


def _rsqrt_newton(d):
    dd = jnp.maximum(d, jnp.float32(1e-30))
    i = lax.bitcast_convert_type(dd, jnp.int32)
    i = jnp.int32(0x5F3759DF) - lax.shift_right_arithmetic(i, jnp.int32(1))
    y = lax.bitcast_convert_type(i, jnp.float32)
    half = jnp.float32(0.5) * dd
    for _ in range(3):
        y = y * (jnp.float32(1.5) - half * y * y)
    return jnp.where(d > 0, y, jnp.float32(0.0))


@functools.partial(
    pl.kernel,
    out_type=jax.ShapeDtypeStruct((NW, NP), jnp.float32),
    mesh=_mesh,
    scratch_types=[
        pltpu.VMEM((EPW,), jnp.int32),
        pltpu.VMEM((EPW,), jnp.float32),
        pltpu.VMEM((NP,), jnp.float32),
    ],
    compiler_params=_sc_params,
)
def _deg(col_hbm, ew_hbm, degp_hbm, colv, ewv, degv):
    cid = lax.axis_index("c")
    sid = lax.axis_index("s")
    wid = sid * NC + cid
    zeros16 = jnp.zeros((16,), jnp.float32)

    @pl.loop(0, NP, step=16)
    def _(i):
        degv[pl.ds(i, 16)] = zeros16

    pltpu.sync_copy(col_hbm.at[wid], colv)
    pltpu.sync_copy(ew_hbm.at[wid], ewv)

    @pl.loop(0, EPW, step=16)
    def _(e):
        sl = pl.ds(e, 16)
        plsc.addupdate_scatter(degv, [colv[sl]], ewv[sl])

    pltpu.sync_copy(degv, degp_hbm.at[wid])


def _degsum_body(p_ref, o_ref):
    o_ref[...] = jnp.sum(p_ref[...], axis=0, keepdims=True)


_degsum = pl.pallas_call(
    _degsum_body, out_shape=jax.ShapeDtypeStruct((1, NP), jnp.float32))


@functools.partial(
    pl.kernel,
    out_type=jax.ShapeDtypeStruct((NC, NP, DH), jnp.float32),
    mesh=_mesh,
    scratch_types=[
        pltpu.VMEM((NCHUNK, K), jnp.int32),
        pltpu.VMEM((NCHUNK, K), jnp.int32),
        pltpu.VMEM((NCHUNK, K), jnp.float32),
        pltpu.VMEM((NP,), jnp.float32),
        pltpu.VMEM((NP,), jnp.float32),
        pltpu.VMEM((K, DH), jnp.float32),
        pltpu.VMEM_SHARED((NP, DH), jnp.float32),
    ],
    compiler_params=_sc_params,
)
def _agg(row_hbm, col_hbm, ew_hbm, degp_hbm, table_hbm, out_hbm,
         rowv, colv, wv, disv, dgv, msgv, acc):
    cid = lax.axis_index("c")
    sid = lax.axis_index("s")
    zeros16 = jnp.zeros((16,), jnp.float32)

    @pl.loop(0, K)
    def _(r):
        for q in range(DH // 16):
            msgv[r, pl.ds(q * 16, 16)] = zeros16

    for j in range(RPT // K):
        pltpu.sync_copy(msgv, acc.at[pl.ds(sid * RPT + j * K, K)])

    pltpu.sync_copy(degp_hbm.at[0], dgv)

    @pl.loop(0, NP, step=16)
    def _(i):
        sl = pl.ds(i, 16)
        disv[sl] = _rsqrt_newton(dgv[sl])

    pltpu.sync_copy(row_hbm.at[sid], rowv)
    pltpu.sync_copy(col_hbm.at[sid], colv)
    pltpu.sync_copy(ew_hbm.at[sid], wv)

    @pl.loop(0, NCHUNK)
    def _(c):
        @pl.loop(0, K, step=16)
        def _(j):
            sl = pl.ds(j, 16)
            dr = plsc.load_gather(disv, [rowv[c, sl]])
            dc = plsc.load_gather(disv, [colv[c, sl]])
            wv[c, sl] = wv[c, sl] * dr * dc

    plsc.subcore_barrier()

    table = table_hbm.at[cid]

    @pl.loop(0, NCHUNK)
    def _(c):
        pltpu.sync_copy(table.at[rowv.at[c]], msgv)
        cvec = jnp.full((16,), c, dtype=jnp.int32)

        @pl.loop(0, K)
        def _(j):
            jvec = jnp.full((16,), j, dtype=jnp.int32)
            wj = plsc.load_gather(wv, [cvec, jvec])
            for q in range(DH // 16):
                sl = pl.ds(q * 16, 16)
                msgv[j, sl] = msgv[j, sl] * wj

        pltpu.sync_copy(msgv, acc.at[colv.at[c]], add=True)

    plsc.subcore_barrier()
    pltpu.sync_copy(acc.at[pl.ds(sid * RPT, RPT)],
                    out_hbm.at[cid, pl.ds(sid * RPT, RPT)])


def _mm_body(x_ref, w_ref, o_ref):
    r = jnp.dot(x_ref[...], w_ref[...], preferred_element_type=jnp.float32)
    o_ref[0] = r[:, :DH]
    o_ref[1] = r[:, DH:]


_mm = pl.pallas_call(
    _mm_body, out_shape=jax.ShapeDtypeStruct((NC, NP, DH), jnp.float32))


def _relu_mm_body(a_ref, b_ref, w_ref, o_ref):
    a = jnp.concatenate([a_ref[0], a_ref[1]], axis=1)
    h = jnp.maximum(a + b_ref[...], 0.0)
    r = jnp.dot(h, w_ref[...], preferred_element_type=jnp.float32)
    o_ref[0] = r[:, :DH]
    o_ref[1] = r[:, DH:]


_relu_mm = pl.pallas_call(
    _relu_mm_body, out_shape=jax.ShapeDtypeStruct((NC, NP, DH), jnp.float32))


def _relu_body(a_ref, b_ref, o_ref):
    a = jnp.concatenate([a_ref[0], a_ref[1]], axis=1)
    o_ref[...] = jnp.maximum(a + b_ref[...], 0.0)


_relu = pl.pallas_call(
    _relu_body, out_shape=jax.ShapeDtypeStruct((NP, D), jnp.float32))


def kernel(x, edge_index, edge_weight, W1, b1, W2, b2):
    row = edge_index[0].astype(jnp.int32)
    col = edge_index[1].astype(jnp.int32)
    loop = jnp.arange(N, dtype=jnp.int32)
    padi = jnp.full((EP - E - N,), NP - 1, dtype=jnp.int32)
    row_f = jnp.concatenate([row, loop, padi])
    col_f = jnp.concatenate([col, loop, padi])
    ew_f = jnp.concatenate([
        edge_weight.astype(jnp.float32),
        jnp.ones((N,), jnp.float32),
        jnp.zeros((EP - E - N,), jnp.float32),
    ])
    row_c = row_f.reshape(NS, NCHUNK, K)
    col_c = col_f.reshape(NS, NCHUNK, K)
    ew_c = ew_f.reshape(NS, NCHUNK, K)
    x_pad = jnp.concatenate([x, jnp.zeros((NP - N, D), x.dtype)])

    degp = _degsum(_deg(col_f.reshape(NW, EPW), ew_f.reshape(NW, EPW)))
    xw1 = _mm(x_pad, W1)
    agg1 = _agg(row_c, col_c, ew_c, degp, xw1)
    xw2 = _relu_mm(agg1, b1.reshape(1, D), W2)
    agg2 = _agg(row_c, col_c, ew_c, degp, xw2)
    out = _relu(agg2, b2.reshape(1, D))
    return out[:N]


# speedup vs baseline: 17.3615x; 1.7959x over previous
import dataclasses
import functools

import jax
import jax.numpy as jnp
from jax import lax
from jax.experimental import pallas as pl
from jax.experimental.pallas import tpu as pltpu
from jax.experimental.pallas import tpu_sc as plsc

N = 10000
NP = 10240
E = 320000
D = 128
DH = 64
NC = 2
NS = 16
NW = NC * NS
EP = 330240
EPW = EP // NW
EPS = EP // NS
K = 80
NCHUNK = EPS // K
NB = 3
RPT = NP // NS

_mesh = plsc.VectorSubcoreMesh(core_axis_name="c", subcore_axis_name="s")

_sc_params = pltpu.CompilerParams(needs_layout_passes=False,
                                  use_tc_tiling_on_sc=False)


def _rsqrt_newton(d):
    dd = jnp.maximum(d, jnp.float32(1e-30))
    i = lax.bitcast_convert_type(dd, jnp.int32)
    i = jnp.int32(0x5F3759DF) - lax.shift_right_arithmetic(i, jnp.int32(1))
    y = lax.bitcast_convert_type(i, jnp.float32)
    half = jnp.float32(0.5) * dd
    for _ in range(3):
        y = y * (jnp.float32(1.5) - half * y * y)
    return jnp.where(d > 0, y, jnp.float32(0.0))


@functools.partial(
    pl.kernel,
    out_type=jax.ShapeDtypeStruct((NW, NP), jnp.float32),
    mesh=_mesh,
    scratch_types=[
        pltpu.VMEM((EPW,), jnp.int32),
        pltpu.VMEM((EPW,), jnp.float32),
        pltpu.VMEM((NP,), jnp.float32),
    ],
    compiler_params=_sc_params,
)
def _deg(col_hbm, ew_hbm, degp_hbm, colv, ewv, degv):
    cid = lax.axis_index("c")
    sid = lax.axis_index("s")
    wid = sid * NC + cid
    zeros16 = jnp.zeros((16,), jnp.float32)

    @pl.loop(0, NP, step=16)
    def _(i):
        degv[pl.ds(i, 16)] = zeros16

    pltpu.sync_copy(col_hbm.at[wid], colv)
    pltpu.sync_copy(ew_hbm.at[wid], ewv)

    @pl.loop(0, EPW, step=16)
    def _(e):
        sl = pl.ds(e, 16)
        plsc.addupdate_scatter(degv, [colv[sl]], ewv[sl])

    pltpu.sync_copy(degv, degp_hbm.at[wid])


def _degsum_body(p_ref, o_ref):
    o_ref[...] = jnp.sum(p_ref[...], axis=0, keepdims=True)


_degsum = pl.pallas_call(
    _degsum_body, out_shape=jax.ShapeDtypeStruct((1, NP), jnp.float32))


@functools.partial(
    pl.kernel,
    out_type=jax.ShapeDtypeStruct((NC, NP, DH), jnp.float32),
    mesh=_mesh,
    scratch_types=[
        pltpu.VMEM((NCHUNK, K), jnp.int32),
        pltpu.VMEM((NCHUNK, K), jnp.int32),
        pltpu.VMEM((NCHUNK, K), jnp.float32),
        pltpu.VMEM((NP,), jnp.float32),
        pltpu.VMEM((NB, K, DH), jnp.float32),
        pltpu.VMEM_SHARED((NP, DH), jnp.float32),
        pltpu.SemaphoreType.DMA((NB,)),
        pltpu.SemaphoreType.DMA((NB,)),
    ],
    compiler_params=_sc_params,
)
def _agg(row_hbm, col_hbm, ew_hbm, degp_hbm, table_hbm, out_hbm,
         rowv, colv, wv, disv, msgv, acc, gsem, ssem):
    cid = lax.axis_index("c")
    sid = lax.axis_index("s")
    zeros16 = jnp.zeros((16,), jnp.float32)

    @pl.loop(0, K)
    def _(r):
        for q in range(DH // 16):
            msgv[0, r, pl.ds(q * 16, 16)] = zeros16

    for j in range(RPT // K):
        pltpu.sync_copy(msgv.at[0], acc.at[pl.ds(sid * RPT + j * K, K)])

    pltpu.sync_copy(degp_hbm.at[0], disv)

    @pl.loop(0, NP, step=16)
    def _(i):
        sl = pl.ds(i, 16)
        disv[sl] = _rsqrt_newton(disv[sl])

    pltpu.sync_copy(row_hbm.at[sid], rowv)
    pltpu.sync_copy(col_hbm.at[sid], colv)
    pltpu.sync_copy(ew_hbm.at[sid], wv)

    @pl.loop(0, NCHUNK)
    def _(c):
        @pl.loop(0, K, step=16)
        def _(j):
            sl = pl.ds(j, 16)
            dr = plsc.load_gather(disv, [rowv[c, sl]])
            dc = plsc.load_gather(disv, [colv[c, sl]])
            wv[c, sl] = wv[c, sl] * dr * dc

    table = table_hbm.at[cid]

    def _gather_start(c, b):
        pltpu.async_copy(table.at[rowv.at[c]], msgv.at[b], gsem.at[b])

    def _gather_wait(c, b):
        pltpu.make_async_copy(table.at[rowv.at[c]], msgv.at[b],
                              gsem.at[b]).wait()

    def _scatter_start(c, b):
        pltpu.async_copy(msgv.at[b], acc.at[colv.at[c]], ssem.at[b],
                         add=True)

    def _scatter_wait(c, b):
        pltpu.make_async_copy(msgv.at[b], acc.at[colv.at[c]],
                              ssem.at[b]).wait()

    _gather_start(0, 0)
    _gather_start(1, 1)
    plsc.subcore_barrier()

    @pl.loop(0, NCHUNK, step=NB)
    def _(c0):
        for b in range(NB):
            c = c0 + b
            _gather_wait(c, b)
            cvec = jnp.full((16,), c, dtype=jnp.int32)

            @pl.loop(0, K)
            def _(j):
                jvec = jnp.full((16,), j, dtype=jnp.int32)
                wj = plsc.load_gather(wv, [cvec, jvec])
                for q in range(DH // 16):
                    sl = pl.ds(q * 16, 16)
                    msgv[b, j, sl] = msgv[b, j, sl] * wj

            _scatter_start(c, b)
            bn = (b + 2) % NB

            @pl.when(jnp.logical_and(c >= 1, c + 2 < NCHUNK))
            def _():
                _scatter_wait(c - 1, bn)

            @pl.when(c + 2 < NCHUNK)
            def _():
                _gather_start(c + 2, bn)

    for b in range(NB):
        _scatter_wait(NCHUNK - NB + b, b)
    plsc.subcore_barrier()
    pltpu.sync_copy(acc.at[pl.ds(sid * RPT, RPT)],
                    out_hbm.at[cid, pl.ds(sid * RPT, RPT)])


def _mm_body(x_ref, w_ref, o_ref):
    r = jnp.dot(x_ref[...], w_ref[...], preferred_element_type=jnp.float32)
    o_ref[0] = r[:, :DH]
    o_ref[1] = r[:, DH:]


_mm = pl.pallas_call(
    _mm_body, out_shape=jax.ShapeDtypeStruct((NC, NP, DH), jnp.float32))


def _relu_mm_body(a_ref, b_ref, w_ref, o_ref):
    a = jnp.concatenate([a_ref[0], a_ref[1]], axis=1)
    h = jnp.maximum(a + b_ref[...], 0.0)
    r = jnp.dot(h, w_ref[...], preferred_element_type=jnp.float32)
    o_ref[0] = r[:, :DH]
    o_ref[1] = r[:, DH:]


_relu_mm = pl.pallas_call(
    _relu_mm_body, out_shape=jax.ShapeDtypeStruct((NC, NP, DH), jnp.float32))


def _relu_body(a_ref, b_ref, o_ref):
    a = jnp.concatenate([a_ref[0], a_ref[1]], axis=1)
    o_ref[...] = jnp.maximum(a + b_ref[...], 0.0)


_relu = pl.pallas_call(
    _relu_body, out_shape=jax.ShapeDtypeStruct((NP, D), jnp.float32))


def kernel(x, edge_index, edge_weight, W1, b1, W2, b2):
    row = edge_index[0].astype(jnp.int32)
    col = edge_index[1].astype(jnp.int32)
    loop = jnp.arange(N, dtype=jnp.int32)
    padi = jnp.full((EP - E - N,), NP - 1, dtype=jnp.int32)
    row_f = jnp.concatenate([row, loop, padi])
    col_f = jnp.concatenate([col, loop, padi])
    ew_f = jnp.concatenate([
        edge_weight.astype(jnp.float32),
        jnp.ones((N,), jnp.float32),
        jnp.zeros((EP - E - N,), jnp.float32),
    ])
    row_c = row_f.reshape(NS, NCHUNK, K)
    col_c = col_f.reshape(NS, NCHUNK, K)
    ew_c = ew_f.reshape(NS, NCHUNK, K)
    x_pad = jnp.concatenate([x, jnp.zeros((NP - N, D), x.dtype)])

    degp = _degsum(_deg(col_f.reshape(NW, EPW), ew_f.reshape(NW, EPW)))
    xw1 = _mm(x_pad, W1)
    agg1 = _agg(row_c, col_c, ew_c, degp, xw1)
    xw2 = _relu_mm(agg1, b1.reshape(1, D), W2)
    agg2 = _agg(row_c, col_c, ew_c, degp, xw2)
    out = _relu(agg2, b2.reshape(1, D))
    return out[:N]
